# Initial kernel scaffold; baseline (speedup 1.0000x reference)
import jax
import jax.numpy as jnp
from jax.experimental import pallas as pl


def kernel(in_feat, edge_index, e_weight, target_node, batch_num_nodes, W1, b1, W2, b2):
    raise NotImplementedError("write your pallas kernel here")


# speedup vs baseline: 2.2127x; 2.2127x over previous
import functools

import jax
import jax.numpy as jnp
from jax import lax
from jax.experimental import pallas as pl
from jax.experimental.pallas import tpu as pltpu
from jax.experimental.pallas import tpu_sc as plsc

_N = 10000
_E = 320000
_D = 128
_C = 16
_G = 4
_NP = 10240
_NT = 32
_EP = _E // _NT
_CH = 2000
_NCH = _EP // _CH
_REG = 6
_DUMP = _NP - 1


def _sc_mesh():
    return plsc.VectorSubcoreMesh(core_axis_name="c", subcore_axis_name="s")


_CP_SC = pltpu.CompilerParams(needs_layout_passes=False)


def _dcount_kernel(src_hbm, dst_hbm, w_hbm, tgt_hbm, part_hbm,
                   hs, hd, af, src_c, dst_c, w_c, tgt_v):
    cc = lax.axis_index("c")
    ss = lax.axis_index("s")
    wid = ss * 2 + cc

    z16 = jnp.zeros((16,), jnp.float32)
    ones = jnp.ones((16,), jnp.float32)

    def zero_hist(i, _):
        af[pl.ds(i * 16, 16)] = z16
        return 0
    lax.fori_loop(0, (4 * _NP) // 16, zero_hist, 0)

    def zero_deg(i, _):
        hs[pl.ds(i * 16, 16)] = z16
        hd[pl.ds(i * 16, 16)] = z16
        return 0
    lax.fori_loop(0, _NP // 16, zero_deg, 0)

    pltpu.sync_copy(tgt_hbm, tgt_v)
    tgv = tgt_v[...]
    t0, t1, t2, t3 = tgv[0], tgv[1], tgv[2], tgv[3]

    base = wid * _EP

    def chunk(ci, _):
        off = base + ci * _CH
        pltpu.sync_copy(src_hbm.at[pl.ds(off, _CH)], src_c)
        pltpu.sync_copy(dst_hbm.at[pl.ds(off, _CH)], dst_c)
        pltpu.sync_copy(w_hbm.at[pl.ds(off, _CH)], w_c)

        def grp(gi, _):
            sl = pl.ds(gi * 16, 16)
            sv = src_c[sl]
            dv = dst_c[sl]
            wv = w_c[sl]
            plsc.addupdate_scatter(hs, [sv], ones)
            plsc.addupdate_scatter(hd, [dv], ones)
            m0 = dv == t0
            m1 = dv == t1
            m2 = dv == t2
            m3 = dv == t3
            m = ((m0 | m1) | (m2 | m3))
            gv = (jnp.where(m1, 1, 0) + jnp.where(m2, 2, 0)
                  + jnp.where(m3, 3, 0)).astype(jnp.int32)
            aidx = gv * _NP + sv
            plsc.addupdate_scatter(af, [aidx], wv, mask=m)
            return 0
        lax.fori_loop(0, _CH // 16, grp, 0)
        return 0
    lax.fori_loop(0, _NCH, chunk, 0)

    pltpu.sync_copy(hs, part_hbm.at[wid, 0])
    pltpu.sync_copy(hd, part_hbm.at[wid, 1])
    for g in range(4):
        pltpu.sync_copy(af.at[pl.ds(g * _NP, _NP)], part_hbm.at[wid, 2 + g])


def _run_dcount(src, dst, w, tgt16):
    k = functools.partial(
        pl.kernel,
        out_type=jax.ShapeDtypeStruct((_NT, _REG, _NP), jnp.float32),
        mesh=_sc_mesh(),
        compiler_params=_CP_SC,
        scratch_types=[
            pltpu.VMEM((_NP,), jnp.float32),
            pltpu.VMEM((_NP,), jnp.float32),
            pltpu.VMEM((4 * _NP,), jnp.float32),
            pltpu.VMEM((_CH,), jnp.int32),
            pltpu.VMEM((_CH,), jnp.int32),
            pltpu.VMEM((_CH,), jnp.float32),
            pltpu.VMEM((16,), jnp.int32),
        ],
    )(_dcount_kernel)
    return k(src, dst, w, tgt16)


def _stats_kernel(part_ref, mark_ref, nsrc_ref, ndst_ref, amat_ref):
    s = jnp.sum(part_ref[...], axis=0)
    nsrc = lax.rsqrt(jnp.maximum(s[0], 1.0))
    ndst = lax.rsqrt(jnp.maximum(s[1], 1.0))
    a = s[2:6]
    nsrc_ref[...] = nsrc
    ndst_ref[...] = ndst
    amat_ref[...] = a * nsrc[None, :]
    tot = jnp.sum(jnp.abs(a), axis=0)
    mark_ref[...] = jnp.where(tot > 0.0, 1.0, 0.0)


def _run_stats(part):
    return pl.pallas_call(
        _stats_kernel,
        out_shape=(
            jax.ShapeDtypeStruct((_NP,), jnp.float32),
            jax.ShapeDtypeStruct((_NP,), jnp.float32),
            jax.ShapeDtypeStruct((_NP,), jnp.float32),
            jax.ShapeDtypeStruct((4, _NP), jnp.float32),
        ),
    )(part)


_NH = _NP // 2
_EPS = _E // 16
_LCAP = _CH + 176


def _gather_kernel(src_hbm, dst_hbm, w_hbm, x_hbm, mark_hbm, nsrc_hbm,
                   ndst_hbm, pagg_hbm,
                   mark_t, nsrc_t, ndst_t, src_c, dst_c, w_c,
                   l_src, l_dst, l_w, rows, idxr, idxw, sem, aggsh):
    cc = lax.axis_index("c")
    ss = lax.axis_index("s")

    pltpu.sync_copy(mark_hbm, mark_t)
    pltpu.sync_copy(nsrc_hbm, nsrc_t)
    pltpu.sync_copy(ndst_hbm, ndst_t)

    z16 = jnp.zeros((16,), jnp.float32)

    def zero_rows(i, _):
        rows[i // 8, pl.ds((i % 8) * 16, 16)] = z16
        return 0
    lax.fori_loop(0, 128 * 8, zero_rows, 0)
    rpt = _NH // 16
    for k in range(rpt // 128):
        pltpu.sync_copy(rows, aggsh.at[pl.ds(ss * rpt + k * 128, 128), :])
    pltpu.sync_copy(rows.at[pl.ds(0, rpt % 128), :],
                    aggsh.at[pl.ds(ss * rpt + (rpt // 128) * 128,
                                   rpt % 128), :])
    plsc.subcore_barrier()

    nbase = cc * _NH
    base = ss * _EPS

    zero16i = jnp.zeros((16,), jnp.int32)

    def chunk(ci, _):
        off = base + ci * _CH
        pltpu.sync_copy(src_hbm.at[pl.ds(off, _CH)], src_c)
        pltpu.sync_copy(dst_hbm.at[pl.ds(off, _CH)], dst_c)
        pltpu.sync_copy(w_hbm.at[pl.ds(off, _CH)], w_c)

        def grp(gi, cnt):
            sl = pl.ds(gi * 16, 16)
            sv = src_c[sl]
            dv = dst_c[sl]
            wv = w_c[sl]
            mkv = plsc.load_gather(mark_t, [dv])
            dloc = dv - nbase
            mine = (dloc >= 0) & (dloc < _NH)
            keep = (mkv > 0.5) & mine
            nsv = plsc.load_gather(nsrc_t, [sv])
            ndv = plsc.load_gather(ndst_t, [dv])
            ws = wv * nsv * ndv
            plsc.store_compressed(l_src.at[pl.ds(cnt, 16)], sv, mask=keep)
            plsc.store_compressed(l_dst.at[pl.ds(cnt, 16)], dloc, mask=keep)
            plsc.store_compressed(l_w.at[pl.ds(cnt, 16)], ws, mask=keep)
            return cnt + jnp.sum(keep.astype(jnp.int32))
        cnt = lax.fori_loop(0, _CH // 16, grp, 0)

        for k in range(8):
            sl = pl.ds(cnt + k * 16, 16)
            l_src[sl] = zero16i
            l_dst[sl] = zero16i
            l_w[sl] = z16

        nch = (cnt + 127) // 128

        def flush(i, _):
            cbase = i * 128
            for k in range(8):
                idxr[pl.ds(k * 16, 16)] = l_src[pl.ds(cbase + k * 16, 16)]
                idxw[pl.ds(k * 16, 16)] = l_dst[pl.ds(cbase + k * 16, 16)]
            pltpu.async_copy(x_hbm.at[idxr], rows, sem).wait()

            def scale(j, _):
                wv = l_w[pl.ds(cbase + j * 16, 16)]
                for lane in range(16):
                    wsc = wv[lane]
                    rr = j * 16 + lane
                    for col in range(8):
                        csl = pl.ds(col * 16, 16)
                        rows[rr, csl] = rows[rr, csl] * wsc
                return 0
            lax.fori_loop(0, 8, scale, 0)
            pltpu.sync_copy(rows, aggsh.at[idxw], add=True)
            return 0
        lax.fori_loop(0, nch, flush, 0)
        return 0
    lax.fori_loop(0, _EPS // _CH, chunk, 0)

    plsc.subcore_barrier()
    pltpu.sync_copy(aggsh.at[pl.ds(ss * rpt, rpt), :],
                    pagg_hbm.at[cc, pl.ds(ss * rpt, rpt), :])


def _run_gather(src, dst, w, x, mark, nsrc, ndst):
    k = functools.partial(
        pl.kernel,
        out_type=jax.ShapeDtypeStruct((2, _NH, _D), jnp.float32),
        mesh=_sc_mesh(),
        compiler_params=_CP_SC,
        scratch_types=[
            pltpu.VMEM((_NP,), jnp.float32),
            pltpu.VMEM((_NP,), jnp.float32),
            pltpu.VMEM((_NP,), jnp.float32),
            pltpu.VMEM((_CH,), jnp.int32),
            pltpu.VMEM((_CH,), jnp.int32),
            pltpu.VMEM((_CH,), jnp.float32),
            pltpu.VMEM((_LCAP,), jnp.int32),
            pltpu.VMEM((_LCAP,), jnp.int32),
            pltpu.VMEM((_LCAP,), jnp.float32),
            pltpu.VMEM((128, _D), jnp.float32),
            pltpu.VMEM((128,), jnp.int32),
            pltpu.VMEM((128,), jnp.int32),
            pltpu.SemaphoreType.DMA,
            pltpu.VMEM_SHARED((_NH, _D), jnp.float32),
        ],
    )(_gather_kernel)
    return k(src, dst, w, x, mark, nsrc, ndst)


def _epilogue_kernel(pagg_ref, w1_ref, b1_ref, w2_ref, b2_ref, amat_ref,
                     ndt_ref, o_ref):
    agg = pagg_ref[...]
    h1 = jnp.dot(agg, w1_ref[...], preferred_element_type=jnp.float32)
    h1 = jnp.maximum(h1 + b1_ref[...], 0.0)
    y = jnp.dot(h1, w2_ref[...], preferred_element_type=jnp.float32)
    out = jnp.dot(amat_ref[...], y, preferred_element_type=jnp.float32)
    o_ref[...] = out * ndt_ref[...] + b2_ref[...]


def _run_epilogue(pagg, w1, b1r, w2, b2r, amat, ndt):
    return pl.pallas_call(
        _epilogue_kernel,
        out_shape=jax.ShapeDtypeStruct((_G, _C), jnp.float32),
    )(pagg, w1, b1r, w2, b2r, amat, ndt)


def kernel(in_feat, edge_index, e_weight, target_node, batch_num_nodes,
           W1, b1, W2, b2):
    offsets = jnp.concatenate(
        [jnp.zeros((1,), batch_num_nodes.dtype),
         jnp.cumsum(batch_num_nodes)])[:-1]
    tgt = (target_node + offsets).astype(jnp.int32)
    tgt16 = jnp.zeros((16,), jnp.int32).at[:4].set(tgt)

    src = edge_index[0]
    dst = edge_index[1]
    part = _run_dcount(src, dst, e_weight, tgt16)
    mark, nsrc, ndst, amat = _run_stats(part)
    pagg = _run_gather(src, dst, e_weight, in_feat, mark, nsrc, ndst)
    pagg = pagg.reshape(_NP, _D)
    ndt = ndst[tgt].reshape(_G, 1)
    return _run_epilogue(pagg, W1, b1.reshape(1, _D), W2,
                         b2.reshape(1, _C), amat, ndt)


# speedup vs baseline: 14.3417x; 6.4817x over previous
import functools

import jax
import jax.numpy as jnp
from jax import lax
from jax.experimental import pallas as pl
from jax.experimental.pallas import tpu as pltpu
from jax.experimental.pallas import tpu_sc as plsc

_N = 10000
_E = 320000
_D = 128
_C = 16
_G = 4
_NP = 10240
_NT = 32
_EP = _E // _NT
_CH = 2000
_NCH = _EP // _CH
_REG = 6
_DUMP = _NP - 1


def _sc_mesh():
    return plsc.VectorSubcoreMesh(core_axis_name="c", subcore_axis_name="s")


_CP_SC = pltpu.CompilerParams(needs_layout_passes=False)


def _dcount_kernel(src_hbm, dst_hbm, w_hbm, tgt_hbm, part_hbm,
                   hs, hd, af, src_c, dst_c, w_c, tgt_v):
    cc = lax.axis_index("c")
    ss = lax.axis_index("s")
    wid = ss * 2 + cc

    z16 = jnp.zeros((16,), jnp.float32)
    ones = jnp.ones((16,), jnp.float32)

    def zero_hist(i, _):
        af[pl.ds(i * 16, 16)] = z16
        return 0
    lax.fori_loop(0, (4 * _NP) // 16, zero_hist, 0)

    def zero_deg(i, _):
        hs[pl.ds(i * 16, 16)] = z16
        hd[pl.ds(i * 16, 16)] = z16
        return 0
    lax.fori_loop(0, _NP // 16, zero_deg, 0)

    pltpu.sync_copy(tgt_hbm, tgt_v)
    tgv = tgt_v[...]
    t0, t1, t2, t3 = tgv[0], tgv[1], tgv[2], tgv[3]

    base = wid * _EP

    def chunk(ci, _):
        off = base + ci * _CH
        pltpu.sync_copy(src_hbm.at[pl.ds(off, _CH)], src_c)
        pltpu.sync_copy(dst_hbm.at[pl.ds(off, _CH)], dst_c)
        pltpu.sync_copy(w_hbm.at[pl.ds(off, _CH)], w_c)

        def grp(gi, _):
            sl = pl.ds(gi * 16, 16)
            sv = src_c[sl]
            dv = dst_c[sl]
            wv = w_c[sl]
            plsc.addupdate_scatter(hs, [sv], ones)
            plsc.addupdate_scatter(hd, [dv], ones)
            m0 = dv == t0
            m1 = dv == t1
            m2 = dv == t2
            m3 = dv == t3
            m = ((m0 | m1) | (m2 | m3))
            gv = (jnp.where(m1, 1, 0) + jnp.where(m2, 2, 0)
                  + jnp.where(m3, 3, 0)).astype(jnp.int32)
            aidx = gv * _NP + sv
            plsc.addupdate_scatter(af, [aidx], wv, mask=m)
            return 0
        lax.fori_loop(0, _CH // 16, grp, 0)
        return 0
    lax.fori_loop(0, _NCH, chunk, 0)

    pltpu.sync_copy(hs, part_hbm.at[wid, 0])
    pltpu.sync_copy(hd, part_hbm.at[wid, 1])
    for g in range(4):
        pltpu.sync_copy(af.at[pl.ds(g * _NP, _NP)], part_hbm.at[wid, 2 + g])


def _run_dcount(src, dst, w, tgt16):
    k = functools.partial(
        pl.kernel,
        out_type=jax.ShapeDtypeStruct((_NT, _REG, _NP), jnp.float32),
        mesh=_sc_mesh(),
        compiler_params=_CP_SC,
        scratch_types=[
            pltpu.VMEM((_NP,), jnp.float32),
            pltpu.VMEM((_NP,), jnp.float32),
            pltpu.VMEM((4 * _NP,), jnp.float32),
            pltpu.VMEM((_CH,), jnp.int32),
            pltpu.VMEM((_CH,), jnp.int32),
            pltpu.VMEM((_CH,), jnp.float32),
            pltpu.VMEM((16,), jnp.int32),
        ],
    )(_dcount_kernel)
    return k(src, dst, w, tgt16)


def _stats_kernel(part_ref, mark_ref, nsrc_ref, ndst_ref, amat_ref):
    s = jnp.sum(part_ref[...], axis=0)
    nsrc = lax.rsqrt(jnp.maximum(s[0], 1.0))
    ndst = lax.rsqrt(jnp.maximum(s[1], 1.0))
    a = s[2:6]
    nsrc_ref[...] = nsrc
    ndst_ref[...] = ndst
    amat_ref[...] = a * nsrc[None, :]
    tot = jnp.sum(jnp.abs(a), axis=0)
    mark_ref[...] = jnp.where(tot > 0.0, 1.0, 0.0)


def _run_stats(part):
    return pl.pallas_call(
        _stats_kernel,
        out_shape=(
            jax.ShapeDtypeStruct((_NP,), jnp.float32),
            jax.ShapeDtypeStruct((_NP,), jnp.float32),
            jax.ShapeDtypeStruct((_NP,), jnp.float32),
            jax.ShapeDtypeStruct((4, _NP), jnp.float32),
        ),
    )(part)


_NH = _NP // 2
_EPS = _E // 16
_LCAP = _CH + 176


def _gather_kernel(src_hbm, dst_hbm, w_hbm, x_hbm, mark_hbm, nsrc_hbm,
                   ndst_hbm, pagg_hbm,
                   mark_t, nsrc_t, ndst_t, src_c, dst_c, w_c,
                   l_src, l_dst, l_w, rows, idxr, idxw, sem, aggsh):
    cc = lax.axis_index("c")
    ss = lax.axis_index("s")

    pltpu.sync_copy(mark_hbm, mark_t)
    pltpu.sync_copy(nsrc_hbm, nsrc_t)
    pltpu.sync_copy(ndst_hbm, ndst_t)

    z16 = jnp.zeros((16,), jnp.float32)

    def zero_rows(i, _):
        rows[i // 8, pl.ds((i % 8) * 16, 16)] = z16
        return 0
    lax.fori_loop(0, 128 * 8, zero_rows, 0)
    rpt = _NH // 16
    for k in range(rpt // 128):
        pltpu.sync_copy(rows, aggsh.at[pl.ds(ss * rpt + k * 128, 128), :])
    pltpu.sync_copy(rows.at[pl.ds(0, rpt % 128), :],
                    aggsh.at[pl.ds(ss * rpt + (rpt // 128) * 128,
                                   rpt % 128), :])
    plsc.subcore_barrier()

    nbase = cc * _NH
    base = ss * _EPS

    zero16i = jnp.zeros((16,), jnp.int32)

    def flush_block(cbase):
        for k in range(8):
            idxr[pl.ds(k * 16, 16)] = l_src[pl.ds(cbase + k * 16, 16)]
            idxw[pl.ds(k * 16, 16)] = l_dst[pl.ds(cbase + k * 16, 16)]
        pltpu.async_copy(x_hbm.at[idxr], rows, sem).wait()

        def scale(j, _):
            wv = l_w[pl.ds(cbase + j * 16, 16)]
            for lane in range(16):
                wsc = wv[lane]
                rr = j * 16 + lane
                for col in range(8):
                    csl = pl.ds(col * 16, 16)
                    rows[rr, csl] = rows[rr, csl] * wsc
            return 0
        lax.fori_loop(0, 8, scale, 0)
        pltpu.sync_copy(rows, aggsh.at[idxw], add=True)

    def chunk(ci, cnt):
        off = base + ci * _CH
        pltpu.sync_copy(src_hbm.at[pl.ds(off, _CH)], src_c)
        pltpu.sync_copy(dst_hbm.at[pl.ds(off, _CH)], dst_c)
        pltpu.sync_copy(w_hbm.at[pl.ds(off, _CH)], w_c)

        def grp(gi, cnt):
            sl = pl.ds(gi * 16, 16)
            dv = dst_c[sl]
            mkv = plsc.load_gather(mark_t, [dv])
            dloc = dv - nbase
            keep = (mkv > 0.5) & (dloc >= 0) & (dloc < _NH)

            def taken():
                sv = src_c[sl]
                wv = w_c[sl]
                nsv = plsc.load_gather(nsrc_t, [sv])
                ndv = plsc.load_gather(ndst_t, [dv])
                ws = wv * nsv * ndv
                plsc.store_compressed(l_src.at[pl.ds(cnt, 16)], sv,
                                      mask=keep)
                plsc.store_compressed(l_dst.at[pl.ds(cnt, 16)], dloc,
                                      mask=keep)
                plsc.store_compressed(l_w.at[pl.ds(cnt, 16)], ws, mask=keep)
                return jnp.sum(keep.astype(jnp.int32))

            kn = lax.cond(jnp.any(keep), taken, lambda: 0)
            return cnt + kn
        cnt = lax.fori_loop(0, _CH // 16, grp, cnt)

        nfl = cnt // 128

        def flush(i, _):
            flush_block(i * 128)
            return 0
        lax.fori_loop(0, nfl, flush, 0)
        tb = nfl * 128
        for k in range(8):
            ssl = pl.ds(tb + k * 16, 16)
            dsl = pl.ds(k * 16, 16)
            l_src[dsl] = l_src[ssl]
            l_dst[dsl] = l_dst[ssl]
            l_w[dsl] = l_w[ssl]
        return cnt - tb
    cnt = lax.fori_loop(0, _EPS // _CH, chunk, 0)

    for k in range(8):
        sl = pl.ds(cnt + k * 16, 16)
        l_src[sl] = zero16i
        l_dst[sl] = zero16i
        l_w[sl] = z16

    @pl.when(cnt > 0)
    def _():
        flush_block(0)

    plsc.subcore_barrier()
    pltpu.sync_copy(aggsh.at[pl.ds(ss * rpt, rpt), :],
                    pagg_hbm.at[cc, pl.ds(ss * rpt, rpt), :])


def _run_gather(src, dst, w, x, mark, nsrc, ndst):
    k = functools.partial(
        pl.kernel,
        out_type=jax.ShapeDtypeStruct((2, _NH, _D), jnp.float32),
        mesh=_sc_mesh(),
        compiler_params=_CP_SC,
        scratch_types=[
            pltpu.VMEM((_NP,), jnp.float32),
            pltpu.VMEM((_NP,), jnp.float32),
            pltpu.VMEM((_NP,), jnp.float32),
            pltpu.VMEM((_CH,), jnp.int32),
            pltpu.VMEM((_CH,), jnp.int32),
            pltpu.VMEM((_CH,), jnp.float32),
            pltpu.VMEM((_LCAP,), jnp.int32),
            pltpu.VMEM((_LCAP,), jnp.int32),
            pltpu.VMEM((_LCAP,), jnp.float32),
            pltpu.VMEM((128, _D), jnp.float32),
            pltpu.VMEM((128,), jnp.int32),
            pltpu.VMEM((128,), jnp.int32),
            pltpu.SemaphoreType.DMA,
            pltpu.VMEM_SHARED((_NH, _D), jnp.float32),
        ],
    )(_gather_kernel)
    return k(src, dst, w, x, mark, nsrc, ndst)


def _epilogue_kernel(pagg_ref, w1_ref, b1_ref, w2_ref, b2_ref, amat_ref,
                     ndt_ref, o_ref):
    agg = pagg_ref[...]
    h1 = jnp.dot(agg, w1_ref[...], preferred_element_type=jnp.float32)
    h1 = jnp.maximum(h1 + b1_ref[...], 0.0)
    y = jnp.dot(h1, w2_ref[...], preferred_element_type=jnp.float32)
    out = jnp.dot(amat_ref[...], y, preferred_element_type=jnp.float32)
    o_ref[...] = out * ndt_ref[...] + b2_ref[...]


def _run_epilogue(pagg, w1, b1r, w2, b2r, amat, ndt):
    return pl.pallas_call(
        _epilogue_kernel,
        out_shape=jax.ShapeDtypeStruct((_G, _C), jnp.float32),
    )(pagg, w1, b1r, w2, b2r, amat, ndt)


def kernel(in_feat, edge_index, e_weight, target_node, batch_num_nodes,
           W1, b1, W2, b2):
    offsets = jnp.concatenate(
        [jnp.zeros((1,), batch_num_nodes.dtype),
         jnp.cumsum(batch_num_nodes)])[:-1]
    tgt = (target_node + offsets).astype(jnp.int32)
    tgt16 = jnp.zeros((16,), jnp.int32).at[:4].set(tgt)

    src = edge_index[0]
    dst = edge_index[1]
    part = _run_dcount(src, dst, e_weight, tgt16)
    mark, nsrc, ndst, amat = _run_stats(part)
    pagg = _run_gather(src, dst, e_weight, in_feat, mark, nsrc, ndst)
    pagg = pagg.reshape(_NP, _D)
    ndt = ndst[tgt].reshape(_G, 1)
    return _run_epilogue(pagg, W1, b1.reshape(1, _D), W2,
                         b2.reshape(1, _C), amat, ndt)


# speedup vs baseline: 14.8662x; 1.0366x over previous
import functools

import jax
import jax.numpy as jnp
from jax import lax
from jax.experimental import pallas as pl
from jax.experimental.pallas import tpu as pltpu
from jax.experimental.pallas import tpu_sc as plsc

_N = 10000
_E = 320000
_D = 128
_C = 16
_G = 4
_NP = 10240
_NT = 32
_EP = _E // _NT
_CH = 2000
_NCH = _EP // _CH
_REG = 6
_DUMP = _NP - 1


def _sc_mesh():
    return plsc.VectorSubcoreMesh(core_axis_name="c", subcore_axis_name="s")


_CP_SC = pltpu.CompilerParams(needs_layout_passes=False)


def _dcount_kernel(src_hbm, dst_hbm, w_hbm, tgt_hbm, part_hbm,
                   hs, hd, af, src_c, dst_c, w_c, tgt_v):
    cc = lax.axis_index("c")
    ss = lax.axis_index("s")
    wid = ss * 2 + cc

    z16 = jnp.zeros((16,), jnp.float32)
    ones = jnp.ones((16,), jnp.float32)

    def zero_hist(i, _):
        af[pl.ds(i * 16, 16)] = z16
        return 0
    lax.fori_loop(0, (4 * _NP) // 16, zero_hist, 0)

    def zero_deg(i, _):
        hs[pl.ds(i * 16, 16)] = z16
        hd[pl.ds(i * 16, 16)] = z16
        return 0
    lax.fori_loop(0, _NP // 16, zero_deg, 0)

    pltpu.sync_copy(tgt_hbm, tgt_v)
    tgv = tgt_v[...]
    t0, t1, t2, t3 = tgv[0], tgv[1], tgv[2], tgv[3]

    base = wid * _EP

    def chunk(ci, _):
        off = base + ci * _CH
        pltpu.sync_copy(src_hbm.at[pl.ds(off, _CH)], src_c)
        pltpu.sync_copy(dst_hbm.at[pl.ds(off, _CH)], dst_c)
        pltpu.sync_copy(w_hbm.at[pl.ds(off, _CH)], w_c)

        def grp(gi, _):
            sl = pl.ds(gi * 16, 16)
            sv = src_c[sl]
            dv = dst_c[sl]
            wv = w_c[sl]
            plsc.addupdate_scatter(hs, [sv], ones)
            plsc.addupdate_scatter(hd, [dv], ones)
            m0 = dv == t0
            m1 = dv == t1
            m2 = dv == t2
            m3 = dv == t3
            m = ((m0 | m1) | (m2 | m3))
            gv = (jnp.where(m1, 1, 0) + jnp.where(m2, 2, 0)
                  + jnp.where(m3, 3, 0)).astype(jnp.int32)
            aidx = gv * _NP + sv
            plsc.addupdate_scatter(af, [aidx], wv, mask=m)
            return 0
        lax.fori_loop(0, _CH // 16, grp, 0)
        return 0
    lax.fori_loop(0, _NCH, chunk, 0)

    pltpu.sync_copy(hs, part_hbm.at[wid, 0])
    pltpu.sync_copy(hd, part_hbm.at[wid, 1])
    for g in range(4):
        pltpu.sync_copy(af.at[pl.ds(g * _NP, _NP)], part_hbm.at[wid, 2 + g])


def _run_dcount(src, dst, w, tgt16):
    k = functools.partial(
        pl.kernel,
        out_type=jax.ShapeDtypeStruct((_NT, _REG, _NP), jnp.float32),
        mesh=_sc_mesh(),
        compiler_params=_CP_SC,
        scratch_types=[
            pltpu.VMEM((_NP,), jnp.float32),
            pltpu.VMEM((_NP,), jnp.float32),
            pltpu.VMEM((4 * _NP,), jnp.float32),
            pltpu.VMEM((_CH,), jnp.int32),
            pltpu.VMEM((_CH,), jnp.int32),
            pltpu.VMEM((_CH,), jnp.float32),
            pltpu.VMEM((16,), jnp.int32),
        ],
    )(_dcount_kernel)
    return k(src, dst, w, tgt16)


def _stats_kernel(part_ref, tab_ref, amat_ref):
    s = jnp.sum(part_ref[...], axis=0)
    nsrc = lax.rsqrt(jnp.maximum(s[0], 1.0))
    ndst = lax.rsqrt(jnp.maximum(s[1], 1.0))
    a = s[2:6]
    amat_ref[...] = a * nsrc[None, :]
    tot = jnp.sum(jnp.abs(a), axis=0)
    tab_ref[0, :] = jnp.where(tot > 0.0, 1.0, 0.0)
    tab_ref[1, :] = nsrc
    tab_ref[2, :] = ndst


def _run_stats(part):
    return pl.pallas_call(
        _stats_kernel,
        out_shape=(
            jax.ShapeDtypeStruct((3, _NP), jnp.float32),
            jax.ShapeDtypeStruct((4, _NP), jnp.float32),
        ),
    )(part)


_NH = _NP // 2
_EPS = _E // 16
_LCAP = _CH + 176


def _gather_kernel(src_hbm, dst_hbm, w_hbm, x_hbm, tab_hbm, pagg_hbm,
                   tab_t, src_a, dst_a, w_a, src_b, dst_b, w_b,
                   l_src, l_dst, l_w, rows, idxr, idxw, sem, sem2, aggsh):
    cc = lax.axis_index("c")
    ss = lax.axis_index("s")

    pltpu.sync_copy(tab_hbm, tab_t)

    z16 = jnp.zeros((16,), jnp.float32)

    def zero_rows(i, _):
        rows[i // 8, pl.ds((i % 8) * 16, 16)] = z16
        return 0
    lax.fori_loop(0, 128 * 8, zero_rows, 0)
    rpt = _NH // 16
    for k in range(rpt // 128):
        pltpu.sync_copy(rows, aggsh.at[pl.ds(ss * rpt + k * 128, 128), :])
    pltpu.sync_copy(rows.at[pl.ds(0, rpt % 128), :],
                    aggsh.at[pl.ds(ss * rpt + (rpt // 128) * 128,
                                   rpt % 128), :])
    plsc.subcore_barrier()

    nbase = cc * _NH
    base = ss * _EPS

    zero16i = jnp.zeros((16,), jnp.int32)

    def flush_block(cbase):
        for k in range(8):
            idxr[pl.ds(k * 16, 16)] = l_src[pl.ds(cbase + k * 16, 16)]
            idxw[pl.ds(k * 16, 16)] = l_dst[pl.ds(cbase + k * 16, 16)]
        pltpu.async_copy(x_hbm.at[idxr], rows, sem).wait()

        def scale(j, _):
            wv = l_w[pl.ds(cbase + j * 16, 16)]
            for lane in range(16):
                wsc = wv[lane]
                rr = j * 16 + lane
                for col in range(8):
                    csl = pl.ds(col * 16, 16)
                    rows[rr, csl] = rows[rr, csl] * wsc
            return 0
        lax.fori_loop(0, 8, scale, 0)
        pltpu.sync_copy(rows, aggsh.at[idxw], add=True)

    def start_load(ci, bufs):
        off = base + ci * _CH
        return (pltpu.async_copy(src_hbm.at[pl.ds(off, _CH)], bufs[0], sem2),
                pltpu.async_copy(dst_hbm.at[pl.ds(off, _CH)], bufs[1], sem2),
                pltpu.async_copy(w_hbm.at[pl.ds(off, _CH)], bufs[2], sem2))

    def scan_chunk(bufs, cnt0):
        src_c, dst_c, w_c = bufs

        def grp(gi, cnt):
            sl = pl.ds(gi * 16, 16)
            dv = dst_c[sl]
            mkv = plsc.load_gather(tab_t, [dv])
            dloc = dv - nbase
            keep = (mkv > 0.5) & (dloc >= 0) & (dloc < _NH)

            def taken():
                sv = src_c[sl]
                wv = w_c[sl]
                nsv = plsc.load_gather(tab_t, [sv + _NP])
                ndv = plsc.load_gather(tab_t, [dv + 2 * _NP])
                ws = wv * nsv * ndv
                plsc.store_compressed(l_src.at[pl.ds(cnt, 16)], sv,
                                      mask=keep)
                plsc.store_compressed(l_dst.at[pl.ds(cnt, 16)], dloc,
                                      mask=keep)
                plsc.store_compressed(l_w.at[pl.ds(cnt, 16)], ws, mask=keep)
                return jnp.sum(keep.astype(jnp.int32))

            kn = lax.cond(jnp.any(keep), taken, lambda: 0)
            return cnt + kn
        cnt = lax.fori_loop(0, _CH // 16, grp, cnt0)

        nfl = cnt // 128

        def flush(i, _):
            flush_block(i * 128)
            return 0
        lax.fori_loop(0, nfl, flush, 0)
        tb = nfl * 128
        for k in range(8):
            ssl = pl.ds(tb + k * 16, 16)
            dsl = pl.ds(k * 16, 16)
            l_src[dsl] = l_src[ssl]
            l_dst[dsl] = l_dst[ssl]
            l_w[dsl] = l_w[ssl]
        return cnt - tb

    bufs_a = (src_a, dst_a, w_a)
    bufs_b = (src_b, dst_b, w_b)
    nchunks = _EPS // _CH
    cnt = 0
    hnd = start_load(0, bufs_a)
    for ci in range(nchunks):
        cur = bufs_a if ci % 2 == 0 else bufs_b
        nxt = bufs_b if ci % 2 == 0 else bufs_a
        for h in hnd:
            h.wait()
        if ci + 1 < nchunks:
            hnd = start_load(ci + 1, nxt)
        cnt = scan_chunk(cur, cnt)

    for k in range(8):
        sl = pl.ds(cnt + k * 16, 16)
        l_src[sl] = zero16i
        l_dst[sl] = zero16i
        l_w[sl] = z16

    @pl.when(cnt > 0)
    def _():
        flush_block(0)

    plsc.subcore_barrier()
    pltpu.sync_copy(aggsh.at[pl.ds(ss * rpt, rpt), :],
                    pagg_hbm.at[cc, pl.ds(ss * rpt, rpt), :])


def _run_gather(src, dst, w, x, tab):
    k = functools.partial(
        pl.kernel,
        out_type=jax.ShapeDtypeStruct((2, _NH, _D), jnp.float32),
        mesh=_sc_mesh(),
        compiler_params=_CP_SC,
        scratch_types=[
            pltpu.VMEM((3 * _NP,), jnp.float32),
            pltpu.VMEM((_CH,), jnp.int32),
            pltpu.VMEM((_CH,), jnp.int32),
            pltpu.VMEM((_CH,), jnp.float32),
            pltpu.VMEM((_CH,), jnp.int32),
            pltpu.VMEM((_CH,), jnp.int32),
            pltpu.VMEM((_CH,), jnp.float32),
            pltpu.VMEM((_LCAP,), jnp.int32),
            pltpu.VMEM((_LCAP,), jnp.int32),
            pltpu.VMEM((_LCAP,), jnp.float32),
            pltpu.VMEM((128, _D), jnp.float32),
            pltpu.VMEM((128,), jnp.int32),
            pltpu.VMEM((128,), jnp.int32),
            pltpu.SemaphoreType.DMA,
            pltpu.SemaphoreType.DMA,
            pltpu.VMEM_SHARED((_NH, _D), jnp.float32),
        ],
    )(_gather_kernel)
    return k(src, dst, w, x, tab)


def _epilogue_kernel(pagg_ref, w1_ref, b1_ref, w2_ref, b2_ref, amat_ref,
                     ndt_ref, o_ref):
    agg = pagg_ref[...]
    h1 = jnp.dot(agg, w1_ref[...], preferred_element_type=jnp.float32)
    h1 = jnp.maximum(h1 + b1_ref[...], 0.0)
    y = jnp.dot(h1, w2_ref[...], preferred_element_type=jnp.float32)
    out = jnp.dot(amat_ref[...], y, preferred_element_type=jnp.float32)
    o_ref[...] = out * ndt_ref[...] + b2_ref[...]


def _run_epilogue(pagg, w1, b1r, w2, b2r, amat, ndt):
    return pl.pallas_call(
        _epilogue_kernel,
        out_shape=jax.ShapeDtypeStruct((_G, _C), jnp.float32),
    )(pagg, w1, b1r, w2, b2r, amat, ndt)


def kernel(in_feat, edge_index, e_weight, target_node, batch_num_nodes,
           W1, b1, W2, b2):
    offsets = jnp.concatenate(
        [jnp.zeros((1,), batch_num_nodes.dtype),
         jnp.cumsum(batch_num_nodes)])[:-1]
    tgt = (target_node + offsets).astype(jnp.int32)
    tgt16 = jnp.zeros((16,), jnp.int32).at[:4].set(tgt)

    src = edge_index[0]
    dst = edge_index[1]
    part = _run_dcount(src, dst, e_weight, tgt16)
    tab, amat = _run_stats(part)
    pagg = _run_gather(src, dst, e_weight, in_feat, tab.reshape(3 * _NP))
    pagg = pagg.reshape(_NP, _D)
    ndt = tab[2, tgt].reshape(_G, 1)
    return _run_epilogue(pagg, W1, b1.reshape(1, _D), W2,
                         b2.reshape(1, _C), amat, ndt)


# speedup vs baseline: 16.3447x; 1.0995x over previous
import functools

import jax
import jax.numpy as jnp
from jax import lax
from jax.experimental import pallas as pl
from jax.experimental.pallas import tpu as pltpu
from jax.experimental.pallas import tpu_sc as plsc

_N = 10000
_E = 320000
_D = 128
_C = 16
_G = 4
_NP = 10240
_NT = 32
_EP = _E // _NT
_CH = 2000
_NCH = _EP // _CH
_REG = 6
_DUMP = _NP - 1


def _sc_mesh():
    return plsc.VectorSubcoreMesh(core_axis_name="c", subcore_axis_name="s")


_CP_SC = pltpu.CompilerParams(needs_layout_passes=False)


def _dcount_kernel(src_hbm, dst_hbm, w_hbm, tgt_hbm, part_hbm,
                   hs, hd, af, src_c, dst_c, w_c, tgt_v):
    cc = lax.axis_index("c")
    ss = lax.axis_index("s")
    wid = ss * 2 + cc

    z16 = jnp.zeros((16,), jnp.float32)
    ones = jnp.ones((16,), jnp.float32)

    def zero_hist(i, _):
        af[pl.ds(i * 16, 16)] = z16
        return 0
    lax.fori_loop(0, (4 * _NP) // 16, zero_hist, 0)

    def zero_deg(i, _):
        hs[pl.ds(i * 16, 16)] = z16
        hd[pl.ds(i * 16, 16)] = z16
        return 0
    lax.fori_loop(0, _NP // 16, zero_deg, 0)

    pltpu.sync_copy(tgt_hbm, tgt_v)
    tgv = tgt_v[...]
    t0, t1, t2, t3 = tgv[0], tgv[1], tgv[2], tgv[3]

    base = wid * _EP

    def chunk(ci, _):
        off = base + ci * _CH
        pltpu.sync_copy(src_hbm.at[pl.ds(off, _CH)], src_c)
        pltpu.sync_copy(dst_hbm.at[pl.ds(off, _CH)], dst_c)
        pltpu.sync_copy(w_hbm.at[pl.ds(off, _CH)], w_c)

        def grp(gi, _):
            sl = pl.ds(gi * 16, 16)
            sv = src_c[sl]
            dv = dst_c[sl]
            wv = w_c[sl]
            plsc.addupdate_scatter(hs, [sv], ones)
            plsc.addupdate_scatter(hd, [dv], ones)
            m0 = dv == t0
            m1 = dv == t1
            m2 = dv == t2
            m3 = dv == t3
            m = ((m0 | m1) | (m2 | m3))
            gv = (jnp.where(m1, 1, 0) + jnp.where(m2, 2, 0)
                  + jnp.where(m3, 3, 0)).astype(jnp.int32)
            aidx = gv * _NP + sv
            plsc.addupdate_scatter(af, [aidx], wv, mask=m)
            return 0
        lax.fori_loop(0, _CH // 16, grp, 0)
        return 0
    lax.fori_loop(0, _NCH, chunk, 0)

    pltpu.sync_copy(hs, part_hbm.at[wid, 0])
    pltpu.sync_copy(hd, part_hbm.at[wid, 1])
    for g in range(4):
        pltpu.sync_copy(af.at[pl.ds(g * _NP, _NP)], part_hbm.at[wid, 2 + g])


def _run_dcount(src, dst, w, tgt16):
    k = functools.partial(
        pl.kernel,
        out_type=jax.ShapeDtypeStruct((_NT, _REG, _NP), jnp.float32),
        mesh=_sc_mesh(),
        compiler_params=_CP_SC,
        scratch_types=[
            pltpu.VMEM((_NP,), jnp.float32),
            pltpu.VMEM((_NP,), jnp.float32),
            pltpu.VMEM((4 * _NP,), jnp.float32),
            pltpu.VMEM((_CH,), jnp.int32),
            pltpu.VMEM((_CH,), jnp.int32),
            pltpu.VMEM((_CH,), jnp.float32),
            pltpu.VMEM((16,), jnp.int32),
        ],
    )(_dcount_kernel)
    return k(src, dst, w, tgt16)


def _stats_kernel(part_ref, tab_ref, amat_ref):
    s = jnp.sum(part_ref[...], axis=0)
    nsrc = lax.rsqrt(jnp.maximum(s[0], 1.0))
    ndst = lax.rsqrt(jnp.maximum(s[1], 1.0))
    a = s[2:6]
    amat_ref[...] = a * nsrc[None, :]
    tot = jnp.sum(jnp.abs(a), axis=0)
    tab_ref[0, :] = jnp.where(tot > 0.0, 1.0, 0.0)
    tab_ref[1, :] = nsrc
    tab_ref[2, :] = ndst


def _run_stats(part):
    return pl.pallas_call(
        _stats_kernel,
        out_shape=(
            jax.ShapeDtypeStruct((3, _NP), jnp.float32),
            jax.ShapeDtypeStruct((4, _NP), jnp.float32),
        ),
    )(part)


_NH = _NP // 2
_EPS = _E // 16
_LCAP = _CH + 176


def _gather_kernel(src_hbm, dst_hbm, w_hbm, x_hbm, tab_hbm, pagg_hbm,
                   tab_t, src_a, dst_a, w_a, src_b, dst_b, w_b,
                   l_src, l_dst, l_w, rows, idxr, idxw, sem, sem2, aggsh):
    cc = lax.axis_index("c")
    ss = lax.axis_index("s")

    pltpu.sync_copy(tab_hbm, tab_t)

    z16 = jnp.zeros((16,), jnp.float32)

    def zero_rows(i, _):
        rows[i // 8, pl.ds((i % 8) * 16, 16)] = z16
        return 0
    lax.fori_loop(0, 128 * 8, zero_rows, 0)
    rpt = _NH // 16
    for k in range(rpt // 128):
        pltpu.sync_copy(rows, aggsh.at[pl.ds(ss * rpt + k * 128, 128), :])
    pltpu.sync_copy(rows.at[pl.ds(0, rpt % 128), :],
                    aggsh.at[pl.ds(ss * rpt + (rpt // 128) * 128,
                                   rpt % 128), :])
    plsc.subcore_barrier()

    nbase = cc * _NH
    base = ss * _EPS

    zero16i = jnp.zeros((16,), jnp.int32)

    def flush_block(cbase):
        for k in range(8):
            idxr[pl.ds(k * 16, 16)] = l_src[pl.ds(cbase + k * 16, 16)]
            idxw[pl.ds(k * 16, 16)] = l_dst[pl.ds(cbase + k * 16, 16)]
        pltpu.async_copy(x_hbm.at[idxr], rows, sem).wait()

        def scale(j, _):
            wv = l_w[pl.ds(cbase + j * 16, 16)]
            for lane in range(16):
                wsc = wv[lane]
                rr = j * 16 + lane
                for col in range(8):
                    csl = pl.ds(col * 16, 16)
                    rows[rr, csl] = rows[rr, csl] * wsc
            return 0
        lax.fori_loop(0, 8, scale, 0)
        pltpu.sync_copy(rows, aggsh.at[idxw], add=True)

    def start_load(ci, bufs):
        off = base + ci * _CH
        return (pltpu.async_copy(src_hbm.at[pl.ds(off, _CH)], bufs[0], sem2),
                pltpu.async_copy(dst_hbm.at[pl.ds(off, _CH)], bufs[1], sem2),
                pltpu.async_copy(w_hbm.at[pl.ds(off, _CH)], bufs[2], sem2))

    def scan_chunk(bufs, cnt0):
        src_c, dst_c, w_c = bufs

        def grp(gi, cnt):
            sl0 = pl.ds(gi * 32, 16)
            sl1 = pl.ds(gi * 32 + 16, 16)
            dv0 = dst_c[sl0]
            dv1 = dst_c[sl1]
            mk0 = plsc.load_gather(tab_t, [dv0])
            mk1 = plsc.load_gather(tab_t, [dv1])
            dl0 = dv0 - nbase
            dl1 = dv1 - nbase
            k0 = (mk0 > 0.5) & (dl0 >= 0) & (dl0 < _NH)
            k1 = (mk1 > 0.5) & (dl1 >= 0) & (dl1 < _NH)

            def compress(sl, dv, dloc, keep, cnt):
                sv = src_c[sl]
                wv = w_c[sl]
                nsv = plsc.load_gather(tab_t, [sv + _NP])
                ndv = plsc.load_gather(tab_t, [dv + 2 * _NP])
                ws = wv * nsv * ndv
                plsc.store_compressed(l_src.at[pl.ds(cnt, 16)], sv,
                                      mask=keep)
                plsc.store_compressed(l_dst.at[pl.ds(cnt, 16)], dloc,
                                      mask=keep)
                plsc.store_compressed(l_w.at[pl.ds(cnt, 16)], ws, mask=keep)
                return cnt + jnp.sum(keep.astype(jnp.int32))

            def taken():
                c1 = compress(sl0, dv0, dl0, k0, cnt)
                return compress(sl1, dv1, dl1, k1, c1)

            return lax.cond(jnp.any(k0 | k1), taken, lambda: cnt)
        cnt = lax.fori_loop(0, _CH // 32, grp, cnt0)

        nfl = cnt // 128

        def flush(i, _):
            flush_block(i * 128)
            return 0
        lax.fori_loop(0, nfl, flush, 0)
        tb = nfl * 128
        for k in range(8):
            ssl = pl.ds(tb + k * 16, 16)
            dsl = pl.ds(k * 16, 16)
            l_src[dsl] = l_src[ssl]
            l_dst[dsl] = l_dst[ssl]
            l_w[dsl] = l_w[ssl]
        return cnt - tb

    bufs_a = (src_a, dst_a, w_a)
    bufs_b = (src_b, dst_b, w_b)
    nchunks = _EPS // _CH
    cnt = 0
    hnd = start_load(0, bufs_a)
    for ci in range(nchunks):
        cur = bufs_a if ci % 2 == 0 else bufs_b
        nxt = bufs_b if ci % 2 == 0 else bufs_a
        for h in hnd:
            h.wait()
        if ci + 1 < nchunks:
            hnd = start_load(ci + 1, nxt)
        cnt = scan_chunk(cur, cnt)

    for k in range(8):
        sl = pl.ds(cnt + k * 16, 16)
        l_src[sl] = zero16i
        l_dst[sl] = zero16i
        l_w[sl] = z16

    @pl.when(cnt > 0)
    def _():
        flush_block(0)

    plsc.subcore_barrier()
    pltpu.sync_copy(aggsh.at[pl.ds(ss * rpt, rpt), :],
                    pagg_hbm.at[cc, pl.ds(ss * rpt, rpt), :])


def _run_gather(src, dst, w, x, tab):
    k = functools.partial(
        pl.kernel,
        out_type=jax.ShapeDtypeStruct((2, _NH, _D), jnp.float32),
        mesh=_sc_mesh(),
        compiler_params=_CP_SC,
        scratch_types=[
            pltpu.VMEM((3 * _NP,), jnp.float32),
            pltpu.VMEM((_CH,), jnp.int32),
            pltpu.VMEM((_CH,), jnp.int32),
            pltpu.VMEM((_CH,), jnp.float32),
            pltpu.VMEM((_CH,), jnp.int32),
            pltpu.VMEM((_CH,), jnp.int32),
            pltpu.VMEM((_CH,), jnp.float32),
            pltpu.VMEM((_LCAP,), jnp.int32),
            pltpu.VMEM((_LCAP,), jnp.int32),
            pltpu.VMEM((_LCAP,), jnp.float32),
            pltpu.VMEM((128, _D), jnp.float32),
            pltpu.VMEM((128,), jnp.int32),
            pltpu.VMEM((128,), jnp.int32),
            pltpu.SemaphoreType.DMA,
            pltpu.SemaphoreType.DMA,
            pltpu.VMEM_SHARED((_NH, _D), jnp.float32),
        ],
    )(_gather_kernel)
    return k(src, dst, w, x, tab)


def _epilogue_kernel(pagg_ref, w1_ref, b1_ref, w2_ref, b2_ref, amat_ref,
                     ndt_ref, o_ref):
    agg = pagg_ref[...]
    h1 = jnp.dot(agg, w1_ref[...], preferred_element_type=jnp.float32)
    h1 = jnp.maximum(h1 + b1_ref[...], 0.0)
    y = jnp.dot(h1, w2_ref[...], preferred_element_type=jnp.float32)
    out = jnp.dot(amat_ref[...], y, preferred_element_type=jnp.float32)
    o_ref[...] = out * ndt_ref[...] + b2_ref[...]


def _run_epilogue(pagg, w1, b1r, w2, b2r, amat, ndt):
    return pl.pallas_call(
        _epilogue_kernel,
        out_shape=jax.ShapeDtypeStruct((_G, _C), jnp.float32),
    )(pagg, w1, b1r, w2, b2r, amat, ndt)


def kernel(in_feat, edge_index, e_weight, target_node, batch_num_nodes,
           W1, b1, W2, b2):
    offsets = jnp.concatenate(
        [jnp.zeros((1,), batch_num_nodes.dtype),
         jnp.cumsum(batch_num_nodes)])[:-1]
    tgt = (target_node + offsets).astype(jnp.int32)
    tgt16 = jnp.zeros((16,), jnp.int32).at[:4].set(tgt)

    src = edge_index[0]
    dst = edge_index[1]
    part = _run_dcount(src, dst, e_weight, tgt16)
    tab, amat = _run_stats(part)
    pagg = _run_gather(src, dst, e_weight, in_feat, tab.reshape(3 * _NP))
    pagg = pagg.reshape(_NP, _D)
    ndt = tab[2, tgt].reshape(_G, 1)
    return _run_epilogue(pagg, W1, b1.reshape(1, _D), W2,
                         b2.reshape(1, _C), amat, ndt)
